# baseline (speedup 1.0000x reference)
import jax
import jax.numpy as jnp
from jax import lax
from jax.experimental import pallas as pl
from jax.experimental.pallas import tpu as pltpu
from jax.experimental.pallas import tpu_sc as plsc

IC = 256
NE = 8192
ED = 64
NT = 8192
HW = 1024
NB = 8
BETA = 0.25
COMMIT_W = 1.0

TB = 4096
_NCH = 16
_CW = NE // _NCH
RB = 1024


def _argmin_body(z_ref, frozen_ref, w_ref, idx_ref, cb_ref, c2_s, fiota_s):
    i = pl.program_id(0)

    @pl.when(i == 0)
    def _():
        cb = lax.dot_general(
            frozen_ref[...], w_ref[...],
            (((1,), (1,)), ((), ())),
            preferred_element_type=jnp.float32,
        )
        cb_ref[...] = cb
        c2_s[0, :] = jnp.sum(cb * cb, axis=1)
        fiota_s[...] = lax.broadcasted_iota(jnp.int32, (1, NE), 1).astype(
            jnp.float32)

    z = z_ref[...]
    zs = z + z
    z2 = jnp.sum(z * z, axis=1, keepdims=True)
    mins = []
    args = []
    for jj in range(_NCH):
        cbj = cb_ref[pl.ds(jj * _CW, _CW), :]
        zc2 = lax.dot_general(
            zs, cbj,
            (((1,), (1,)), ((), ())),
            preferred_element_type=jnp.float32,
        )
        d = (z2 + c2_s[0:1, pl.ds(jj * _CW, _CW)]) - zc2
        m = jnp.min(d, axis=1, keepdims=True)
        a = jnp.min(
            jnp.where(d == m, fiota_s[0:1, pl.ds(jj * _CW, _CW)], float(NE)),
            axis=1, keepdims=True)
        mins.append(m)
        args.append(a)
    while len(mins) > 1:
        nm, na = [], []
        for k in range(0, len(mins), 2):
            ml, mr = mins[k], mins[k + 1]
            al, ar = args[k], args[k + 1]
            nm.append(jnp.minimum(ml, mr))
            na.append(jnp.where(ml <= mr, al, ar))
        mins, args = nm, na
    idx_ref[0, pl.ds(i * TB, TB)] = args[0][:, 0].astype(jnp.int32)


def _rotate_body(z_ref, zq_ref, rot_ref, loss_ref):
    i = pl.program_id(0)
    e = z_ref[...]
    t = zq_ref[...]
    ns = jnp.sqrt(jnp.sum(e * e, axis=1, keepdims=True))
    nt = jnp.sqrt(jnp.sum(t * t, axis=1, keepdims=True))
    u = e / jnp.clip(ns, 1e-6, None)
    q = t / jnp.clip(nt, 1e-6, None)
    w = u + q
    w = w / jnp.clip(jnp.sqrt(jnp.sum(w * w, axis=1, keepdims=True)), 1e-6, None)
    ew = jnp.sum(e * w, axis=1, keepdims=True)
    eu = jnp.sum(e * u, axis=1, keepdims=True)
    rot = e - 2.0 * ew * w + 2.0 * eu * q
    rot_ref[...] = rot * (nt / jnp.clip(ns, 1e-6, None))
    diff = e - t
    part = jnp.sum(diff * diff).reshape(1, 1)

    @pl.when(i == 0)
    def _():
        loss_ref[...] = part

    @pl.when(i > 0)
    def _():
        loss_ref[...] = loss_ref[...] + part

    @pl.when(i == NT // RB - 1)
    def _():
        mean = loss_ref[...] / float(NT * IC)
        loss_ref[...] = (mean + mean * BETA) * COMMIT_W


_argmin_call = pl.pallas_call(
    _argmin_body,
    grid=(NT // TB,),
    in_specs=[
        pl.BlockSpec((TB, IC), lambda i: (i, 0)),
        pl.BlockSpec((NE, ED), lambda i: (0, 0)),
        pl.BlockSpec((IC, ED), lambda i: (0, 0)),
    ],
    out_specs=[
        pl.BlockSpec((1, NT), lambda i: (0, 0)),
        pl.BlockSpec((NE, IC), lambda i: (0, 0)),
    ],
    out_shape=[
        jax.ShapeDtypeStruct((1, NT), jnp.int32),
        jax.ShapeDtypeStruct((NE, IC), jnp.float32),
    ],
    scratch_shapes=[
        pltpu.VMEM((1, NE), jnp.float32),
        pltpu.VMEM((1, NE), jnp.float32),
    ],
)

_rotate_call = pl.pallas_call(
    _rotate_body,
    grid=(NT // RB,),
    in_specs=[
        pl.BlockSpec((RB, IC), lambda i: (i, 0)),
        pl.BlockSpec((RB, IC), lambda i: (i, 0)),
    ],
    out_specs=[
        pl.BlockSpec((RB, IC), lambda i: (i, 0)),
        pl.BlockSpec((1, 1), lambda i: (0, 0)),
    ],
    out_shape=[
        jax.ShapeDtypeStruct((NT, IC), jnp.float32),
        jax.ShapeDtypeStruct((1, 1), jnp.float32),
    ],
)

_SC_CORES = 2
_SC_SUBCORES = 16
_NW = _SC_CORES * _SC_SUBCORES
_BPW = NT // _NW


_HB = _BPW // 2


def _gather_body(cb_hbm, idx_hbm, out_hbm, idx_v, rows_v0, rows_v1,
                 sem0, sem1, semw0, semw1):
    wid = lax.axis_index("s") * _SC_CORES + lax.axis_index("c")
    base = wid * _BPW
    pltpu.sync_copy(idx_hbm.at[pl.ds(base, _BPW)], idx_v)
    g0 = pltpu.async_copy(cb_hbm.at[idx_v.at[pl.ds(0, _HB)]], rows_v0, sem0)
    g1 = pltpu.async_copy(cb_hbm.at[idx_v.at[pl.ds(_HB, _HB)]], rows_v1, sem1)
    g0.wait()
    w0 = pltpu.async_copy(rows_v0, out_hbm.at[pl.ds(base, _HB)], semw0)
    g1.wait()
    w1 = pltpu.async_copy(rows_v1, out_hbm.at[pl.ds(base + _HB, _HB)], semw1)
    w0.wait()
    w1.wait()


def _gather_call(cb, idx):
    call = pl.kernel(
        _gather_body,
        out_type=jax.ShapeDtypeStruct((NT, IC), jnp.float32),
        mesh=plsc.VectorSubcoreMesh(
            core_axis_name="c", subcore_axis_name="s",
            num_cores=_SC_CORES, num_subcores=_SC_SUBCORES,
        ),
        scratch_types=[
            pltpu.VMEM((_BPW,), jnp.int32),
            pltpu.VMEM((_HB, IC), jnp.float32),
            pltpu.VMEM((_HB, IC), jnp.float32),
            pltpu.SemaphoreType.DMA,
            pltpu.SemaphoreType.DMA,
            pltpu.SemaphoreType.DMA,
            pltpu.SemaphoreType.DMA,
        ],
    )
    return call(cb, idx)


@jax.jit
def kernel(z, frozen_codebook, W):
    z = z.astype(jnp.float32)
    z_flat = jnp.transpose(z, (0, 2, 3, 1)).reshape(NT, IC)
    idx2d, cb = _argmin_call(z_flat, frozen_codebook, W)
    idx = idx2d.reshape(NT)
    z_q_flat = _gather_call(cb, idx)
    rot, loss2d = _rotate_call(z_flat, z_q_flat)
    z_q = jnp.transpose(rot.reshape(NB, 32, 32, IC), (0, 3, 1, 2))
    return (z_q, loss2d[0, 0], idx)


# speedup vs baseline: 1.2277x; 1.2277x over previous
import jax
import jax.numpy as jnp
from jax import lax
from jax.experimental import pallas as pl
from jax.experimental.pallas import tpu as pltpu
from jax.experimental.pallas import tpu_sc as plsc

IC = 256
NE = 8192
ED = 64
NT = 8192
HW = 1024
NB = 8
BETA = 0.25
COMMIT_W = 1.0

TB = 2048
_NCH = 8
_CW = NE // _NCH
RB = 1024


def _argmin_body(z_ref, frozen_ref, w_ref, idx_ref, cb_ref, c2_s, fiota_s):
    i = pl.program_id(0)

    @pl.when(i == 0)
    def _():
        cb = lax.dot_general(
            frozen_ref[...], w_ref[...],
            (((1,), (1,)), ((), ())),
            preferred_element_type=jnp.float32,
        )
        cb_ref[...] = cb
        c2_s[0, :] = jnp.sum(cb * cb, axis=1)
        fiota_s[...] = lax.broadcasted_iota(jnp.int32, (1, NE), 1).astype(
            jnp.float32)

    z = z_ref[...]
    zs = z + z
    z2 = jnp.sum(z * z, axis=1, keepdims=True)
    mins = []
    args = []
    for jj in range(_NCH):
        cbj = cb_ref[pl.ds(jj * _CW, _CW), :]
        zc2 = lax.dot_general(
            zs, cbj,
            (((1,), (1,)), ((), ())),
            preferred_element_type=jnp.float32,
        )
        d = (z2 + c2_s[0:1, pl.ds(jj * _CW, _CW)]) - zc2
        m = jnp.min(d, axis=1, keepdims=True)
        a = jnp.min(
            jnp.where(d == m, fiota_s[0:1, pl.ds(jj * _CW, _CW)], float(NE)),
            axis=1, keepdims=True)
        mins.append(m)
        args.append(a)
    while len(mins) > 1:
        nm, na = [], []
        for k in range(0, len(mins), 2):
            ml, mr = mins[k], mins[k + 1]
            al, ar = args[k], args[k + 1]
            nm.append(jnp.minimum(ml, mr))
            na.append(jnp.where(ml <= mr, al, ar))
        mins, args = nm, na
    idx_ref[0, pl.ds(i * TB, TB)] = args[0][:, 0].astype(jnp.int32)


def _rotate_body(z_ref, zq_ref, rot_ref, loss_ref):
    i = pl.program_id(0)
    e = z_ref[...]
    t = zq_ref[...]
    ns = jnp.sqrt(jnp.sum(e * e, axis=1, keepdims=True))
    nt = jnp.sqrt(jnp.sum(t * t, axis=1, keepdims=True))
    u = e / jnp.clip(ns, 1e-6, None)
    q = t / jnp.clip(nt, 1e-6, None)
    w = u + q
    w = w / jnp.clip(jnp.sqrt(jnp.sum(w * w, axis=1, keepdims=True)), 1e-6, None)
    ew = jnp.sum(e * w, axis=1, keepdims=True)
    eu = jnp.sum(e * u, axis=1, keepdims=True)
    rot = e - 2.0 * ew * w + 2.0 * eu * q
    rot_ref[...] = rot * (nt / jnp.clip(ns, 1e-6, None))
    diff = e - t
    part = jnp.sum(diff * diff).reshape(1, 1)

    @pl.when(i == 0)
    def _():
        loss_ref[...] = part

    @pl.when(i > 0)
    def _():
        loss_ref[...] = loss_ref[...] + part

    @pl.when(i == NT // RB - 1)
    def _():
        mean = loss_ref[...] / float(NT * IC)
        loss_ref[...] = (mean + mean * BETA) * COMMIT_W


_argmin_call = pl.pallas_call(
    _argmin_body,
    grid=(NT // TB,),
    in_specs=[
        pl.BlockSpec((TB, IC), lambda i: (i, 0)),
        pl.BlockSpec((NE, ED), lambda i: (0, 0)),
        pl.BlockSpec((IC, ED), lambda i: (0, 0)),
    ],
    out_specs=[
        pl.BlockSpec((1, NT), lambda i: (0, 0)),
        pl.BlockSpec((NE, IC), lambda i: (0, 0)),
    ],
    out_shape=[
        jax.ShapeDtypeStruct((1, NT), jnp.int32),
        jax.ShapeDtypeStruct((NE, IC), jnp.float32),
    ],
    scratch_shapes=[
        pltpu.VMEM((1, NE), jnp.float32),
        pltpu.VMEM((1, NE), jnp.float32),
    ],
)

_rotate_call = pl.pallas_call(
    _rotate_body,
    grid=(NT // RB,),
    in_specs=[
        pl.BlockSpec((RB, IC), lambda i: (i, 0)),
        pl.BlockSpec((RB, IC), lambda i: (i, 0)),
    ],
    out_specs=[
        pl.BlockSpec((RB, IC), lambda i: (i, 0)),
        pl.BlockSpec((1, 1), lambda i: (0, 0)),
    ],
    out_shape=[
        jax.ShapeDtypeStruct((NT, IC), jnp.float32),
        jax.ShapeDtypeStruct((1, 1), jnp.float32),
    ],
)

_SC_CORES = 2
_SC_SUBCORES = 16
_NW = _SC_CORES * _SC_SUBCORES
_BPW = NT // _NW


_HB = _BPW // 2


def _gather_body(cb_hbm, idx_hbm, out_hbm, idx_v, rows_v0, rows_v1,
                 sem0, sem1, semw0, semw1):
    wid = lax.axis_index("s") * _SC_CORES + lax.axis_index("c")
    base = wid * _BPW
    pltpu.sync_copy(idx_hbm.at[pl.ds(base, _BPW)], idx_v)
    g0 = pltpu.async_copy(cb_hbm.at[idx_v.at[pl.ds(0, _HB)]], rows_v0, sem0)
    g1 = pltpu.async_copy(cb_hbm.at[idx_v.at[pl.ds(_HB, _HB)]], rows_v1, sem1)
    g0.wait()
    w0 = pltpu.async_copy(rows_v0, out_hbm.at[pl.ds(base, _HB)], semw0)
    g1.wait()
    w1 = pltpu.async_copy(rows_v1, out_hbm.at[pl.ds(base + _HB, _HB)], semw1)
    w0.wait()
    w1.wait()


def _gather_call(cb, idx):
    call = pl.kernel(
        _gather_body,
        out_type=jax.ShapeDtypeStruct((NT, IC), jnp.float32),
        mesh=plsc.VectorSubcoreMesh(
            core_axis_name="c", subcore_axis_name="s",
            num_cores=_SC_CORES, num_subcores=_SC_SUBCORES,
        ),
        scratch_types=[
            pltpu.VMEM((_BPW,), jnp.int32),
            pltpu.VMEM((_HB, IC), jnp.float32),
            pltpu.VMEM((_HB, IC), jnp.float32),
            pltpu.SemaphoreType.DMA,
            pltpu.SemaphoreType.DMA,
            pltpu.SemaphoreType.DMA,
            pltpu.SemaphoreType.DMA,
        ],
    )
    return call(cb, idx)


@jax.jit
def kernel(z, frozen_codebook, W):
    z = z.astype(jnp.float32)
    z_flat = jnp.transpose(z, (0, 2, 3, 1)).reshape(NT, IC)
    idx2d, cb = _argmin_call(z_flat, frozen_codebook, W)
    idx = idx2d.reshape(NT)
    z_q_flat = _gather_call(cb, idx)
    rot, loss2d = _rotate_call(z_flat, z_q_flat)
    z_q = jnp.transpose(rot.reshape(NB, 32, 32, IC), (0, 3, 1, 2))
    return (z_q, loss2d[0, 0], idx)


# speedup vs baseline: 1.2424x; 1.0120x over previous
import jax
import jax.numpy as jnp
from jax import lax
from jax.experimental import pallas as pl
from jax.experimental.pallas import tpu as pltpu
from jax.experimental.pallas import tpu_sc as plsc

IC = 256
NE = 8192
ED = 64
NT = 8192
HW = 1024
NB = 8
BETA = 0.25
COMMIT_W = 1.0

TB = 2048
_NCH = 16
_CW = NE // _NCH
RB = 2048


def _argmin_body(z_ref, frozen_ref, w_ref, idx_ref, cb_ref, c2_s, fiota_s):
    i = pl.program_id(0)

    @pl.when(i == 0)
    def _():
        cb = lax.dot_general(
            frozen_ref[...], w_ref[...],
            (((1,), (1,)), ((), ())),
            preferred_element_type=jnp.float32,
        )
        cb_ref[...] = cb
        c2_s[0, :] = jnp.sum(cb * cb, axis=1)
        fiota_s[...] = lax.broadcasted_iota(jnp.int32, (1, NE), 1).astype(
            jnp.float32)

    z = z_ref[...]
    zs = z + z
    z2 = jnp.sum(z * z, axis=1, keepdims=True)
    mins = []
    args = []
    for jj in range(_NCH):
        cbj = cb_ref[pl.ds(jj * _CW, _CW), :]
        zc2 = lax.dot_general(
            zs, cbj,
            (((1,), (1,)), ((), ())),
            preferred_element_type=jnp.float32,
        )
        d = (z2 + c2_s[0:1, pl.ds(jj * _CW, _CW)]) - zc2
        m = jnp.min(d, axis=1, keepdims=True)
        a = jnp.min(
            jnp.where(d == m, fiota_s[0:1, pl.ds(jj * _CW, _CW)], float(NE)),
            axis=1, keepdims=True)
        mins.append(m)
        args.append(a)
    while len(mins) > 1:
        nm, na = [], []
        for k in range(0, len(mins), 2):
            ml, mr = mins[k], mins[k + 1]
            al, ar = args[k], args[k + 1]
            nm.append(jnp.minimum(ml, mr))
            na.append(jnp.where(ml <= mr, al, ar))
        mins, args = nm, na
    idx_ref[0, pl.ds(i * TB, TB)] = args[0][:, 0].astype(jnp.int32)


def _rotate_body(z_ref, zq_ref, rot_ref, loss_ref):
    i = pl.program_id(0)
    e = z_ref[...]
    t = zq_ref[...]
    ns = jnp.sqrt(jnp.sum(e * e, axis=1, keepdims=True))
    nt = jnp.sqrt(jnp.sum(t * t, axis=1, keepdims=True))
    u = e / jnp.clip(ns, 1e-6, None)
    q = t / jnp.clip(nt, 1e-6, None)
    w = u + q
    w = w / jnp.clip(jnp.sqrt(jnp.sum(w * w, axis=1, keepdims=True)), 1e-6, None)
    ew = jnp.sum(e * w, axis=1, keepdims=True)
    eu = jnp.sum(e * u, axis=1, keepdims=True)
    rot = e - 2.0 * ew * w + 2.0 * eu * q
    rot_ref[...] = rot * (nt / jnp.clip(ns, 1e-6, None))
    diff = e - t
    part = jnp.sum(diff * diff).reshape(1, 1)

    @pl.when(i == 0)
    def _():
        loss_ref[...] = part

    @pl.when(i > 0)
    def _():
        loss_ref[...] = loss_ref[...] + part

    @pl.when(i == NT // RB - 1)
    def _():
        mean = loss_ref[...] / float(NT * IC)
        loss_ref[...] = (mean + mean * BETA) * COMMIT_W


_argmin_call = pl.pallas_call(
    _argmin_body,
    grid=(NT // TB,),
    in_specs=[
        pl.BlockSpec((TB, IC), lambda i: (i, 0)),
        pl.BlockSpec((NE, ED), lambda i: (0, 0)),
        pl.BlockSpec((IC, ED), lambda i: (0, 0)),
    ],
    out_specs=[
        pl.BlockSpec((1, NT), lambda i: (0, 0)),
        pl.BlockSpec((NE, IC), lambda i: (0, 0)),
    ],
    out_shape=[
        jax.ShapeDtypeStruct((1, NT), jnp.int32),
        jax.ShapeDtypeStruct((NE, IC), jnp.float32),
    ],
    scratch_shapes=[
        pltpu.VMEM((1, NE), jnp.float32),
        pltpu.VMEM((1, NE), jnp.float32),
    ],
)

_rotate_call = pl.pallas_call(
    _rotate_body,
    grid=(NT // RB,),
    in_specs=[
        pl.BlockSpec((RB, IC), lambda i: (i, 0)),
        pl.BlockSpec((RB, IC), lambda i: (i, 0)),
    ],
    out_specs=[
        pl.BlockSpec((RB, IC), lambda i: (i, 0)),
        pl.BlockSpec((1, 1), lambda i: (0, 0)),
    ],
    out_shape=[
        jax.ShapeDtypeStruct((NT, IC), jnp.float32),
        jax.ShapeDtypeStruct((1, 1), jnp.float32),
    ],
)

_SC_CORES = 2
_SC_SUBCORES = 16
_NW = _SC_CORES * _SC_SUBCORES
_BPW = NT // _NW


_HB = _BPW // 2


def _gather_body(cb_hbm, idx_hbm, out_hbm, idx_v, rows_v0, rows_v1,
                 sem0, sem1, semw0, semw1):
    wid = lax.axis_index("s") * _SC_CORES + lax.axis_index("c")
    base = wid * _BPW
    pltpu.sync_copy(idx_hbm.at[pl.ds(base, _BPW)], idx_v)
    g0 = pltpu.async_copy(cb_hbm.at[idx_v.at[pl.ds(0, _HB)]], rows_v0, sem0)
    g1 = pltpu.async_copy(cb_hbm.at[idx_v.at[pl.ds(_HB, _HB)]], rows_v1, sem1)
    g0.wait()
    w0 = pltpu.async_copy(rows_v0, out_hbm.at[pl.ds(base, _HB)], semw0)
    g1.wait()
    w1 = pltpu.async_copy(rows_v1, out_hbm.at[pl.ds(base + _HB, _HB)], semw1)
    w0.wait()
    w1.wait()


def _gather_call(cb, idx):
    call = pl.kernel(
        _gather_body,
        out_type=jax.ShapeDtypeStruct((NT, IC), jnp.float32),
        mesh=plsc.VectorSubcoreMesh(
            core_axis_name="c", subcore_axis_name="s",
            num_cores=_SC_CORES, num_subcores=_SC_SUBCORES,
        ),
        scratch_types=[
            pltpu.VMEM((_BPW,), jnp.int32),
            pltpu.VMEM((_HB, IC), jnp.float32),
            pltpu.VMEM((_HB, IC), jnp.float32),
            pltpu.SemaphoreType.DMA,
            pltpu.SemaphoreType.DMA,
            pltpu.SemaphoreType.DMA,
            pltpu.SemaphoreType.DMA,
        ],
    )
    return call(cb, idx)


@jax.jit
def kernel(z, frozen_codebook, W):
    z = z.astype(jnp.float32)
    z_flat = jnp.transpose(z, (0, 2, 3, 1)).reshape(NT, IC)
    idx2d, cb = _argmin_call(z_flat, frozen_codebook, W)
    idx = idx2d.reshape(NT)
    z_q_flat = _gather_call(cb, idx)
    rot, loss2d = _rotate_call(z_flat, z_q_flat)
    z_q = jnp.transpose(rot.reshape(NB, 32, 32, IC), (0, 3, 1, 2))
    return (z_q, loss2d[0, 0], idx)
